# baseline (speedup 1.0000x reference)
import functools

import jax
import jax.numpy as jnp
from jax import lax
from jax.experimental import pallas as pl
from jax.experimental.pallas import tpu as pltpu
from jax.experimental.pallas import tpu_sc as plsc

N_VOCAB = 100000
N_EMBED = 128
BATCH = 16384

NC = 2
NS = 16
NW = NC * NS
B_PER_W = BATCH // NW
CHUNK = 128
C = B_PER_W // CHUNK
NBUF = 4
SBUF = 2

_mesh = plsc.VectorSubcoreMesh(core_axis_name="c", subcore_axis_name="s")


@functools.partial(
    pl.kernel,
    mesh=_mesh,
    out_type=(
        jax.ShapeDtypeStruct((BATCH, N_EMBED), jnp.float32),
        jax.ShapeDtypeStruct((BATCH, N_EMBED), jnp.float32),
    ),
    scratch_types=[
        pltpu.VMEM((C, CHUNK), jnp.int32),
        pltpu.VMEM((C, CHUNK), jnp.int32),
        pltpu.VMEM((NBUF, CHUNK, N_EMBED), jnp.float32),
        pltpu.VMEM_SHARED((NS, SBUF, CHUNK, N_EMBED), jnp.float32),
        pltpu.SemaphoreType.DMA,
        pltpu.SemaphoreType.DMA,
    ]
    + [pltpu.SemaphoreType.DMA] * (NBUF + 2 * SBUF),
)
def _gather2(iw_hbm, ow_hbm, in_embed, out_embed, o1_hbm, o2_hbm,
             idx1, idx2, rows, shr, si1, si2, *sems):
    sid = lax.axis_index("s")
    wid = sid * NC + lax.axis_index("c")
    gs = sems[:NBUF]
    ms = sems[NBUF:NBUF + SBUF]
    ws = sems[NBUF + SBUF:]
    idxs = (idx1, idx2)
    tables = (in_embed, out_embed)
    outs = (o1_hbm, o2_hbm)

    c1 = pltpu.async_copy(iw_hbm.at[wid], idx1, si1)
    c2 = pltpu.async_copy(ow_hbm.at[wid], idx2, si2)

    total = 2 * C
    gd = []
    md = []
    wd = []

    def fire(k):
        t, j = divmod(k, C)
        if k == 0:
            c1.wait()
        if k == C:
            c2.wait()
        gd.append(
            pltpu.async_copy(tables[t].at[idxs[t].at[j]], rows.at[k % NBUF],
                             gs[k % NBUF]))

    for k in range(NBUF):
        fire(k)
    for k in range(total):
        t, j = divmod(k, C)
        gd[k].wait()
        if k >= SBUF:
            wd[k - SBUF].wait()
        md.append(
            pltpu.async_copy(rows.at[k % NBUF], shr.at[sid, k % SBUF],
                             ms[k % SBUF]))
        md[k].wait()
        base = wid * B_PER_W + j * CHUNK
        wd.append(
            pltpu.async_copy(shr.at[sid, k % SBUF],
                             outs[t].at[pl.ds(base, CHUNK)], ws[k % SBUF]))
        if k + NBUF < total:
            fire(k + NBUF)
    for k in range(total - SBUF, total):
        wd[k].wait()


def kernel(input_words, output_words, in_embed, out_embed):
    iw = input_words.astype(jnp.int32).reshape(NW, C, CHUNK)
    ow = output_words.astype(jnp.int32).reshape(NW, C, CHUNK)
    return _gather2(iw, ow, in_embed, out_embed)


# speedup vs baseline: 1.0451x; 1.0451x over previous
import functools

import jax
import jax.numpy as jnp
from jax import lax
from jax.experimental import pallas as pl
from jax.experimental.pallas import tpu as pltpu
from jax.experimental.pallas import tpu_sc as plsc

N_VOCAB = 100000
N_EMBED = 128
BATCH = 16384

NC = 2
NS = 16
NW = NC * NS
B_PER_W = BATCH // NW
CHUNK = 128
C = B_PER_W // CHUNK
NBUF = 7

_mesh = plsc.VectorSubcoreMesh(core_axis_name="c", subcore_axis_name="s")


@functools.partial(
    pl.kernel,
    mesh=_mesh,
    out_type=(
        jax.ShapeDtypeStruct((BATCH, N_EMBED), jnp.float32),
        jax.ShapeDtypeStruct((BATCH, N_EMBED), jnp.float32),
    ),
    scratch_types=[
        pltpu.VMEM((C, CHUNK), jnp.int32),
        pltpu.VMEM((C, CHUNK), jnp.int32),
        pltpu.VMEM((NBUF, CHUNK, N_EMBED), jnp.float32),
        pltpu.SemaphoreType.DMA,
        pltpu.SemaphoreType.DMA,
    ]
    + [pltpu.SemaphoreType.DMA] * (2 * NBUF),
)
def _gather2(iw_hbm, ow_hbm, in_embed, out_embed, o1_hbm, o2_hbm,
             idx1, idx2, rows, si1, si2, *sems):
    wid = lax.axis_index("s") * NC + lax.axis_index("c")
    gs = sems[:NBUF]
    ws = sems[NBUF:]
    idxs = (idx1, idx2)
    tables = (in_embed, out_embed)
    outs = (o1_hbm, o2_hbm)

    c1 = pltpu.async_copy(iw_hbm.at[wid], idx1, si1)
    c2 = pltpu.async_copy(ow_hbm.at[wid], idx2, si2)

    total = 2 * C
    gd = []
    wd = []

    def fire(k):
        t, j = divmod(k, C)
        if k == 0:
            c1.wait()
        if k == C:
            c2.wait()
        gd.append(
            pltpu.async_copy(tables[t].at[idxs[t].at[j]], rows.at[k % NBUF],
                             gs[k % NBUF]))

    for k in range(NBUF):
        fire(k)
    for k in range(total):
        t, j = divmod(k, C)
        gd[k].wait()
        base = wid * B_PER_W + j * CHUNK
        wd.append(
            pltpu.async_copy(rows.at[k % NBUF], outs[t].at[pl.ds(base, CHUNK)],
                             ws[k % NBUF]))
        if k + NBUF < total:
            wd[k].wait()
            fire(k + NBUF)
    for k in range(max(0, total - NBUF), total):
        wd[k].wait()


def kernel(input_words, output_words, in_embed, out_embed):
    iw = input_words.astype(jnp.int32).reshape(NW, C, CHUNK)
    ow = output_words.astype(jnp.int32).reshape(NW, C, CHUNK)
    return _gather2(iw, ow, in_embed, out_embed)
